# Initial kernel scaffold; baseline (speedup 1.0000x reference)
import jax
import jax.numpy as jnp
from jax.experimental import pallas as pl


def kernel(sentence, context, W):
    raise NotImplementedError("write your pallas kernel here")


# speedup vs baseline: 2.8996x; 2.8996x over previous
import functools

import jax
import jax.numpy as jnp
from jax import lax
from jax.experimental import pallas as pl
from jax.experimental.pallas import tpu as pltpu
from jax.experimental.pallas import tpu_sc as plsc

NC = 2
NS = 16
NW = NC * NS
LANES = 16
CHUNK = 128


def _logsigmoid(z):
    u = jnp.exp(-jnp.abs(z))
    s = u / (u + 2.0)
    s2 = s * s
    p = jnp.float32(1.0 / 13.0)
    for c in (1.0 / 11.0, 1.0 / 9.0, 1.0 / 7.0, 1.0 / 5.0, 1.0 / 3.0, 1.0):
        p = p * s2 + jnp.float32(c)
    log1p = 2.0 * s * p
    return jnp.minimum(z, 0.0) - log1p


def _make_sc_kernel(n_tokens, embed):
    per_worker = n_tokens // NW
    n_chunks = per_worker // CHUNK
    mesh = plsc.VectorSubcoreMesh(core_axis_name="c", subcore_axis_name="s")

    @functools.partial(
        pl.kernel,
        out_type=jax.ShapeDtypeStruct((NW, LANES), jnp.float32),
        mesh=mesh,
        compiler_params=pltpu.CompilerParams(needs_layout_passes=False,
                                             use_tc_tiling_on_sc=False),
        scratch_types=[
            pltpu.VMEM((CHUNK,), jnp.int32),
            pltpu.VMEM((CHUNK, embed), jnp.float32),
            pltpu.VMEM((CHUNK, embed), jnp.float32),
            pltpu.VMEM((LANES * LANES,), jnp.float32),
            pltpu.VMEM((LANES,), jnp.float32),
            pltpu.SemaphoreType.DMA,
        ],
    )
    def sc_kernel(sent_hbm, ctx_hbm, w_hbm, out_hbm, idx_v, ctx_v, wrows_v,
                  ps_v, acc_v, sem):
        wid = lax.axis_index("s") * NC + lax.axis_index("c")
        base0 = wid * per_worker
        acc_v[...] = jnp.zeros((LANES,), jnp.float32)

        def chunk_body(k, carry):
            base = base0 + k * CHUNK
            pltpu.sync_copy(sent_hbm.at[pl.ds(base, CHUNK)], idx_v)
            gat = pltpu.async_copy(w_hbm.at[idx_v], wrows_v, sem)
            pltpu.sync_copy(ctx_hbm.at[pl.ds(base, CHUNK)], ctx_v)
            gat.wait()

            def group_body(g, carry2):
                t0 = g * LANES
                for t in range(LANES):
                    row = t0 + t
                    p = (wrows_v[row, pl.ds(0, LANES)]
                         * ctx_v[row, pl.ds(0, LANES)])
                    for e in range(1, embed // LANES):
                        p = p + (wrows_v[row, pl.ds(e * LANES, LANES)]
                                 * ctx_v[row, pl.ds(e * LANES, LANES)])
                    ps_v[pl.ds(t * LANES, LANES)] = p
                rows16 = lax.iota(jnp.int32, LANES) * LANES
                z = plsc.load_gather(ps_v, [rows16])
                for j in range(1, LANES):
                    z = z + plsc.load_gather(ps_v, [rows16 + j])
                acc_v[...] = acc_v[...] + _logsigmoid(z)
                return carry2

            return lax.fori_loop(0, CHUNK // LANES, group_body, carry)

        lax.fori_loop(0, n_chunks, chunk_body, 0)
        pltpu.sync_copy(acc_v, out_hbm.at[wid])

    return sc_kernel


def kernel(sentence, context, W):
    b, l = sentence.shape
    embed = W.shape[1]
    n_tokens = b * l
    sent_flat = sentence.reshape(n_tokens)
    ctx_flat = context.reshape(n_tokens, embed)
    partials = _make_sc_kernel(n_tokens, embed)(sent_flat, ctx_flat, W)
    return (-jnp.sum(partials) / b).astype(jnp.float32)
